# baseline (speedup 1.0000x reference)
import functools

import jax
import jax.numpy as jnp
from jax import lax
from jax.experimental import pallas as pl
from jax.experimental.pallas import tpu as pltpu
from jax.experimental.pallas import tpu_sc as plsc

_N = 819200
_NOFF = 16384
_NW = 32
_C = _N // _NW
_Q = _C // 4
_O = _NOFF // _NW
_LANES = 16

_mesh = plsc.VectorSubcoreMesh(core_axis_name="c", subcore_axis_name="s")


@functools.partial(
    pl.kernel,
    mesh=_mesh,
    out_type=(
        jax.ShapeDtypeStruct((2 * _NOFF + 1,), jnp.int32),
        jax.ShapeDtypeStruct((2 * _N,), jnp.float32),
    ),
    scratch_types=[
        pltpu.VMEM((_C,), jnp.float32),
        pltpu.VMEM((_C,), jnp.float32),
        pltpu.VMEM((_O,), jnp.int32),
        pltpu.VMEM((_O,), jnp.int32),
        pltpu.VMEM((_LANES,), jnp.int32),
    ] + [pltpu.SemaphoreType.DMA] * 11,
)
def _sc_psw_off(off0, off1, psw0, psw1,
                out_off, out_psw,
                b_p0, b_p1, ob0, ob1, tail_buf,
                go0, go1, g0, g1, g2, g3, g4, g5, g6, g7, ssem):
    wid = lax.axis_index("s") * 2 + lax.axis_index("c")
    ib = wid * _C
    ob = wid * _O

    oh0 = pltpu.async_copy(off0.at[pl.ds(ob, _O)], ob0, go0)
    oh1 = pltpu.async_copy(off1.at[pl.ds(ob, _O)], ob1, go1)

    gsems = [g0, g1, g2, g3, g4, g5, g6, g7]
    chunks = []
    for q in range(4):
        for u, (src, buf, base) in enumerate([
                (psw0, b_p0, 0), (psw1, b_p1, _N)]):
            chunks.append((src.at[pl.ds(ib + q * _Q, _Q)],
                           buf.at[pl.ds(q * _Q, _Q)],
                           out_psw.at[pl.ds(base + ib + q * _Q, _Q)],
                           gsems[q * 2 + u]))
    gathers = [pltpu.async_copy(s, b, g) for s, b, _, g in chunks]

    scatters = []

    oh0.wait()
    scatters.append(pltpu.async_copy(ob0, out_off.at[pl.ds(ob, _O)], ssem))
    oh1.wait()
    for j in range(_O // _LANES):
        sl = pl.ds(j * _LANES, _LANES)
        ob1[sl] = ob1[sl] + jnp.int32(_N)
    scatters.append(pltpu.async_copy(ob1, out_off.at[pl.ds(_NOFF + ob, _O)], ssem))

    @pl.when(wid == _NW - 1)
    def _():
        tail_buf[...] = jnp.full((_LANES,), 2 * _N, jnp.int32)
        pltpu.sync_copy(tail_buf.at[pl.ds(0, 1)], out_off.at[pl.ds(2 * _NOFF, 1)])

    for gh, (_, buf, dst, _) in zip(gathers, chunks):
        gh.wait()
        scatters.append(pltpu.async_copy(buf, dst, ssem))
    for sh in scatters:
        sh.wait()


_BLK = 204800
_NB = _N // _BLK


def _tc_concat_body(i0_ref, i1_ref, out_ref):
    i = pl.program_id(0)

    @pl.when(i < _NB)
    def _():
        out_ref[...] = i0_ref[...]

    @pl.when(i >= _NB)
    def _():
        out_ref[...] = i1_ref[...]


def _tc_concat(idx0, idx1):
    return pl.pallas_call(
        _tc_concat_body,
        grid=(2 * _NB,),
        in_specs=[
            pl.BlockSpec((_BLK,), lambda i: (jnp.minimum(i, _NB - 1),)),
            pl.BlockSpec((_BLK,), lambda i: (jnp.maximum(i - _NB, 0),)),
        ],
        out_specs=pl.BlockSpec((_BLK,), lambda i: (i,)),
        out_shape=jax.ShapeDtypeStruct((2 * _N,), jnp.int32),
    )(idx0, idx1)


def kernel(indices_0, indices_1, offsets_0, offsets_1,
           per_sample_weights_0, per_sample_weights_1):
    out_off, out_psw = _sc_psw_off(offsets_0, offsets_1,
                                   per_sample_weights_0, per_sample_weights_1)
    out_idx = _tc_concat(indices_0, indices_1)
    return out_idx, out_off, out_psw


# speedup vs baseline: 1.0282x; 1.0282x over previous
import functools

import jax
import jax.numpy as jnp
from jax import lax
from jax.experimental import pallas as pl
from jax.experimental.pallas import tpu as pltpu
from jax.experimental.pallas import tpu_sc as plsc

_N = 819200
_NOFF = 16384
_NW = 32
_C = _N // _NW
_H = _C // 2
_O = _NOFF // _NW
_LANES = 16

_mesh = plsc.VectorSubcoreMesh(core_axis_name="c", subcore_axis_name="s")


@functools.partial(
    pl.kernel,
    mesh=_mesh,
    out_type=(
        jax.ShapeDtypeStruct((2 * _NOFF + 1,), jnp.int32),
        jax.ShapeDtypeStruct((2 * _N,), jnp.float32),
    ),
    scratch_types=[
        pltpu.VMEM((_C,), jnp.float32),
        pltpu.VMEM((_C,), jnp.float32),
        pltpu.VMEM((_O,), jnp.int32),
        pltpu.VMEM((_O,), jnp.int32),
        pltpu.VMEM((_LANES,), jnp.int32),
    ] + [pltpu.SemaphoreType.DMA] * 7,
)
def _sc_psw_off(off0, off1, psw0, psw1,
                out_off, out_psw,
                b_p0, b_p1, ob0, ob1, tail_buf,
                go0, go1, g0, g1, g2, g3, ssem):
    wid = lax.axis_index("s") * 2 + lax.axis_index("c")
    ib = wid * _C
    ob = wid * _O

    oh0 = pltpu.async_copy(off0.at[pl.ds(ob, _O)], ob0, go0)
    oh1 = pltpu.async_copy(off1.at[pl.ds(ob, _O)], ob1, go1)

    halves = []
    for h in range(2):
        for u, (src, buf, base, g) in enumerate([
                (psw0, b_p0, 0, (g0, g2)), (psw1, b_p1, _N, (g1, g3))]):
            halves.append((src.at[pl.ds(ib + h * _H, _H)],
                           buf.at[pl.ds(h * _H, _H)],
                           out_psw.at[pl.ds(base + ib + h * _H, _H)],
                           g[h]))
    gathers = [pltpu.async_copy(s, b, g) for s, b, _, g in halves]

    scatters = []

    oh0.wait()
    scatters.append(pltpu.async_copy(ob0, out_off.at[pl.ds(ob, _O)], ssem))
    oh1.wait()
    for j in range(_O // _LANES):
        sl = pl.ds(j * _LANES, _LANES)
        ob1[sl] = ob1[sl] + jnp.int32(_N)
    scatters.append(pltpu.async_copy(ob1, out_off.at[pl.ds(_NOFF + ob, _O)], ssem))

    @pl.when(wid == _NW - 1)
    def _():
        tail_buf[...] = jnp.full((_LANES,), 2 * _N, jnp.int32)
        pltpu.sync_copy(tail_buf.at[pl.ds(0, 1)], out_off.at[pl.ds(2 * _NOFF, 1)])

    for gh, (_, buf, dst, _) in zip(gathers, halves):
        gh.wait()
        scatters.append(pltpu.async_copy(buf, dst, ssem))
    for sh in scatters:
        sh.wait()


_BLK = 204800
_NB = _N // _BLK


def _tc_concat_body(i0_ref, i1_ref, out_ref):
    i = pl.program_id(0)

    @pl.when(i < _NB)
    def _():
        out_ref[...] = i0_ref[...]

    @pl.when(i >= _NB)
    def _():
        out_ref[...] = i1_ref[...]


def _tc_concat(idx0, idx1):
    return pl.pallas_call(
        _tc_concat_body,
        grid=(2 * _NB,),
        in_specs=[
            pl.BlockSpec((_BLK,), lambda i: (jnp.minimum(i, _NB - 1),)),
            pl.BlockSpec((_BLK,), lambda i: (jnp.maximum(i - _NB, 0),)),
        ],
        out_specs=pl.BlockSpec((_BLK,), lambda i: (i,)),
        out_shape=jax.ShapeDtypeStruct((2 * _N,), jnp.int32),
    )(idx0, idx1)


def kernel(indices_0, indices_1, offsets_0, offsets_1,
           per_sample_weights_0, per_sample_weights_1):
    out_off, out_psw = _sc_psw_off(offsets_0, offsets_1,
                                   per_sample_weights_0, per_sample_weights_1)
    out_idx = _tc_concat(indices_0, indices_1)
    return out_idx, out_off, out_psw
